# Initial kernel scaffold; baseline (speedup 1.0000x reference)
import jax
import jax.numpy as jnp
from jax.experimental import pallas as pl


def kernel(frames):
    raise NotImplementedError("write your pallas kernel here")


# speedup vs baseline: 1.5270x; 1.5270x over previous
import jax
import jax.numpy as jnp
from jax.experimental import pallas as pl
from jax.experimental.pallas import tpu as pltpu

_H = 256
_W = 256


def _pack_body(in_ref, slow_ref, fast_ref):
    fast_ref[...] = in_ref[...]
    k = pl.program_id(0)
    j = jax.lax.rem(k, 16)
    off = jax.lax.div(63 * j, 15) - 4 * j
    slow_ref[...] = in_ref[pl.ds(off, 1), :, :]


def _pack(frames_flat):
    n_groups = frames_flat.shape[0] // 4
    return pl.pallas_call(
        _pack_body,
        grid=(n_groups,),
        in_specs=[pl.BlockSpec((4, _H, _W), lambda k: (k, 0, 0))],
        out_specs=[
            pl.BlockSpec((1, _H, _W), lambda k: (k, 0, 0)),
            pl.BlockSpec((4, _H, _W), lambda k: (k, 0, 0)),
        ],
        out_shape=[
            jax.ShapeDtypeStruct((n_groups, _H, _W), jnp.float32),
            jax.ShapeDtypeStruct((n_groups * 4, _H, _W), jnp.float32),
        ],
        compiler_params=pltpu.CompilerParams(
            dimension_semantics=("arbitrary",),
        ),
    )(frames_flat)


def kernel(frames):
    c, t, h, w = frames.shape
    flat = frames.reshape(c * t, h, w)
    slow, fast = _pack(flat)
    return (
        slow.reshape(c, t // 4, h, w),
        fast.reshape(c, t, h, w),
    )
